# baseline (speedup 1.0000x reference)
import functools

import jax
import jax.numpy as jnp
from jax import lax
from jax.experimental import pallas as pl
from jax.experimental.pallas import tpu as pltpu
from jax.experimental.pallas import tpu_sc as plsc

B = 16384
D = 64
H = 128

_NC = 2
_NS = 16
_NW = _NC * _NS
_CHUNK = 128
_ROWS = B // _CHUNK
_RPW = _ROWS // _NW


def _sc_gather3(p_idx2, w_idx2, l_idx2, prompt_emb, video_emb):
    mesh = plsc.VectorSubcoreMesh(
        core_axis_name="c", subcore_axis_name="s",
        num_cores=_NC, num_subcores=_NS)

    out_t = jax.ShapeDtypeStruct((_ROWS, _CHUNK, D), jnp.float32)

    @functools.partial(
        pl.kernel,
        out_type=(out_t, out_t, out_t),
        mesh=mesh,
        scratch_types=[
            pltpu.VMEM((_RPW, _CHUNK), jnp.int32),
            pltpu.VMEM((_RPW, _CHUNK), jnp.int32),
            pltpu.VMEM((_RPW, _CHUNK), jnp.int32),
            pltpu.VMEM((_RPW, _CHUNK, D), jnp.float32),
            pltpu.VMEM((_RPW, _CHUNK, D), jnp.float32),
            pltpu.VMEM((_RPW, _CHUNK, D), jnp.float32),
            pltpu.SemaphoreType.DMA,
        ],
        compiler_params=pltpu.CompilerParams(use_tc_tiling_on_sc=False),
    )
    def gather_kernel(p_idx_hbm, w_idx_hbm, l_idx_hbm, pemb_hbm, vemb_hbm,
                      out_p, out_w, out_l,
                      pidx_v, widx_v, lidx_v, prow_v, wrow_v, lrow_v, sem):
        wid = lax.axis_index("s") * _NC + lax.axis_index("c")
        base = wid * _RPW
        pltpu.sync_copy(p_idx_hbm.at[pl.ds(base, _RPW)], pidx_v)
        pltpu.sync_copy(w_idx_hbm.at[pl.ds(base, _RPW)], widx_v)
        pltpu.sync_copy(l_idx_hbm.at[pl.ds(base, _RPW)], lidx_v)
        copies = []
        for j in range(_RPW):
            copies.append(pltpu.async_copy(
                pemb_hbm.at[pidx_v.at[j]], prow_v.at[j], sem))
            copies.append(pltpu.async_copy(
                vemb_hbm.at[widx_v.at[j]], wrow_v.at[j], sem))
            copies.append(pltpu.async_copy(
                vemb_hbm.at[lidx_v.at[j]], lrow_v.at[j], sem))
        for c in copies:
            c.wait()
        pltpu.sync_copy(prow_v, out_p.at[pl.ds(base, _RPW)])
        pltpu.sync_copy(wrow_v, out_w.at[pl.ds(base, _RPW)])
        pltpu.sync_copy(lrow_v, out_l.at[pl.ds(base, _RPW)])

    return gather_kernel(p_idx2, w_idx2, l_idx2, prompt_emb, video_emb)


_BLK = 4096
_INV_SQRT2 = 0.7071067811865476


def _gelu(x):
    return 0.5 * x * (1.0 + lax.erf(x * _INV_SQRT2))


def _mlp_body(p_ref, vw_ref, vl_ref, w1a_ref, w1b_ref, b1_ref,
              w2_ref, b2_ref, w3_ref, b3_ref, rw_ref, rl_ref):
    p = p_ref[...]
    pa = jnp.dot(p, w1a_ref[...], preferred_element_type=jnp.float32)
    b1 = b1_ref[...]
    w2 = w2_ref[...]
    b2 = b2_ref[...]
    w3 = w3_ref[...]
    b3 = b3_ref[0, 0]
    for v_ref, out_ref in ((vw_ref, rw_ref), (vl_ref, rl_ref)):
        h = pa + jnp.dot(v_ref[...], w1b_ref[...],
                         preferred_element_type=jnp.float32) + b1
        h = _gelu(h)
        h = jnp.dot(h, w2, preferred_element_type=jnp.float32) + b2
        h = _gelu(h)
        out_ref[...] = jnp.sum(h * w3, axis=1) + b3


def _mlp_head(p, vw, vl, W1, b1, W2, b2, W3, b3):
    w1a = W1[:D]
    w1b = W1[D:]
    b1r = b1.reshape(1, H)
    b2r = b2.reshape(1, H)
    w3r = W3.reshape(1, H)
    b3r = b3.reshape(1, 1)
    grid = (B // _BLK,)
    row_spec = pl.BlockSpec((_BLK, D), lambda i: (i, 0))
    full = lambda shape: pl.BlockSpec(shape, lambda i: (0,) * len(shape))
    return pl.pallas_call(
        _mlp_body,
        grid=grid,
        in_specs=[
            row_spec, row_spec, row_spec,
            full((D, H)), full((D, H)), full((1, H)),
            full((H, H)), full((1, H)), full((1, H)), full((1, 1)),
        ],
        out_specs=[pl.BlockSpec((_BLK,), lambda i: (i,)),
                   pl.BlockSpec((_BLK,), lambda i: (i,))],
        out_shape=[jax.ShapeDtypeStruct((B,), jnp.float32),
                   jax.ShapeDtypeStruct((B,), jnp.float32)],
        compiler_params=pltpu.CompilerParams(
            dimension_semantics=("parallel",)),
    )(p, vw, vl, w1a, w1b, b1r, W2, b2r, w3r, b3r)


def kernel(prompt_idx, preferred_idx, rejected_idx, video_emb, prompt_emb,
           W1, b1, W2, b2, W3, b3):
    p_idx2 = prompt_idx.reshape(_ROWS, _CHUNK)
    w_idx2 = preferred_idx.reshape(_ROWS, _CHUNK)
    l_idx2 = rejected_idx.reshape(_ROWS, _CHUNK)
    p3, vw3, vl3 = _sc_gather3(p_idx2, w_idx2, l_idx2, prompt_emb, video_emb)
    p = p3.reshape(B, D)
    vw = vw3.reshape(B, D)
    vl = vl3.reshape(B, D)
    r_w, r_l = _mlp_head(p, vw, vl, W1, b1, W2, b2, W3, b3)
    return (r_w, r_l)
